# baseline (speedup 1.0000x reference)
import jax
import jax.numpy as jnp
import numpy as np
from jax.experimental import pallas as pl

_B, _N, _F = 1024, 128, 16
_K = 7
_BB = 8
_EPS = 1e-3


def _fold_bn(p):
    g, b, m, v = p
    s = g * jax.lax.rsqrt(v + _EPS)
    return s, b - m * s


def _edge_conv(E, fts, Wtop, Wbot, b1, W2, b2, W3, b3, Wsc, bsc):
    rows = _BB * _N
    inf = jnp.float32(np.inf)
    diag = (jax.lax.broadcasted_iota(jnp.int32, (rows, _N), 0) % _N
            == jax.lax.broadcasted_iota(jnp.int32, (rows, _N), 1))
    E = jnp.where(diag, inf, E)

    f32 = jnp.float32
    u = jnp.dot(fts, Wbot)
    base = jnp.dot(fts, Wtop) - u + b1
    ub = u.astype(jnp.bfloat16)

    acc = jnp.zeros((rows, W3.shape[1]), jnp.float32)
    for k in range(_K):
        mval = jnp.min(E, axis=1, keepdims=True)
        ohb = E == mval
        E = jnp.where(ohb, inf, E)
        oh = ohb.astype(jnp.bfloat16)
        g = jnp.concatenate(
            [jnp.dot(oh[s * _N:(s + 1) * _N], ub[s * _N:(s + 1) * _N],
                     preferred_element_type=f32)
             for s in range(_BB)], axis=0)
        h = jnp.maximum(base + g, 0.0).astype(jnp.bfloat16)
        h = jnp.maximum(jnp.dot(h, W2, preferred_element_type=f32) + b2,
                        0.0).astype(jnp.bfloat16)
        h = jnp.dot(h, W3, preferred_element_type=f32) + b3
        acc = acc + jnp.maximum(h, 0.0)
    fts_out = acc * jnp.float32(1.0 / _K)
    sc = jnp.dot(fts, Wsc) + bsc
    return jnp.maximum(sc + fts_out, 0.0)


def _body(f_ref, mavg_ref, s0_ref, t0_ref,
          wt0_ref, wb0_ref, b10_ref, w20_ref, b20_ref, w30_ref, b30_ref,
          wsc0_ref, bsc0_ref,
          wt1_ref, wb1_ref, b11_ref, w21_ref, b21_ref, w31_ref, b31_ref,
          wsc1_ref, bsc1_ref,
          fcw_ref, fcb_ref, ow_ref, ob_ref, out_ref):
    rows = _BB * _N
    feats = jnp.reshape(f_ref[...], (rows, _F))
    reduced = jnp.sum(feats, axis=1, keepdims=True)
    mask = (reduced != 0.0).astype(jnp.float32)
    shift = 1e9 * (1.0 - mask)

    fts = feats * s0_ref[...] + t0_ref[...]

    e_list = []
    for s in range(_BB):
        fT = f_ref[s].T
        rT = jnp.sum(fT, axis=0, keepdims=True)
        srow = 1e9 * (rT == 0.0).astype(jnp.float32)
        px = fT[0:1] * jnp.cos(fT[1:2]) + srow
        py = fT[0:1] * jnp.sin(fT[1:2]) + srow
        ptsT = jnp.concatenate([px, py], axis=0)
        mm = jax.lax.dot_general(ptsT, ptsT, (((0,), (0,)), ((), ())))
        e_list.append(px * px + py * py - 2.0 * mm)
    E0 = jnp.concatenate(e_list, axis=0)

    fts = _edge_conv(E0, fts,
                     wt0_ref[...], wb0_ref[...], b10_ref[...],
                     w20_ref[...], b20_ref[...], w30_ref[...], b30_ref[...],
                     wsc0_ref[...], bsc0_ref[...])

    ones_c = jnp.ones((1, fts.shape[1]), jnp.float32)
    e_list = []
    for s in range(_BB):
        P = fts[s * _N:(s + 1) * _N] + shift[s * _N:(s + 1) * _N]
        mm = jax.lax.dot_general(P, P, (((1,), (1,)), ((), ())))
        ra = jax.lax.dot_general(ones_c, P * P, (((1,), (1,)), ((), ())))
        e_list.append(ra - 2.0 * mm)
    E1 = jnp.concatenate(e_list, axis=0)

    fts = _edge_conv(E1, fts,
                     wt1_ref[...], wb1_ref[...], b11_ref[...],
                     w21_ref[...], b21_ref[...], w31_ref[...], b31_ref[...],
                     wsc1_ref[...], bsc1_ref[...])

    fts = fts * mask
    pool = jnp.dot(mavg_ref[...], fts)
    x = jnp.maximum(jnp.dot(pool, fcw_ref[...]) + fcb_ref[...], 0.0)
    logits = jnp.dot(x, ow_ref[...]) + ob_ref[...]
    z = logits - jnp.max(logits, axis=1, keepdims=True)
    e = jnp.exp(z)
    out_ref[...] = e / jnp.sum(e, axis=1, keepdims=True)


def _prep_weights(params):
    s0, t0 = _fold_bn(params["bn0"])
    ws = [s0.reshape(1, -1), t0.reshape(1, -1)]
    for layer in params["layers"]:
        w1, w2, w3 = layer["ws"]
        c_in = w1.shape[0] // 2
        s1, t1 = _fold_bn(layer["bns"][0])
        s2, t2 = _fold_bn(layer["bns"][1])
        s3, t3 = _fold_bn(layer["bns"][2])
        ssc, tsc = _fold_bn(layer["bnsc"])
        bf = jnp.bfloat16
        ws += [w1[:c_in] * s1, w1[c_in:] * s1, t1.reshape(1, -1),
               (w2 * s2).astype(bf), t2.reshape(1, -1),
               (w3 * s3).astype(bf), t3.reshape(1, -1),
               layer["wsc"] * ssc, tsc.reshape(1, -1)]
    ws += [params["fc_w"], params["fc_b"].reshape(1, -1),
           params["out_w"], params["out_b"].reshape(1, -1)]
    return ws


def kernel(features, params):
    ws = _prep_weights(params)
    mavg = jnp.asarray(np.kron(np.eye(_BB, dtype=np.float32),
                               np.full((1, _N), 1.0 / _N, np.float32)))
    full = lambda a: pl.BlockSpec(a.shape, lambda i: (0,) * a.ndim)
    out = pl.pallas_call(
        _body,
        grid=(_B // _BB,),
        in_specs=[pl.BlockSpec((_BB, _N, _F), lambda i: (i, 0, 0)),
                  full(mavg)] + [full(a) for a in ws],
        out_specs=pl.BlockSpec((_BB, 5), lambda i: (i, 0)),
        out_shape=jax.ShapeDtypeStruct((_B, 5), jnp.float32),
    )(features, mavg, *ws)
    return out


# speedup vs baseline: 1.0038x; 1.0038x over previous
import jax
import jax.numpy as jnp
import numpy as np
from jax.experimental import pallas as pl

_B, _N, _F = 1024, 128, 16
_K = 7
_BB = 8
_EPS = 1e-3


def _fold_bn(p):
    g, b, m, v = p
    s = g * jax.lax.rsqrt(v + _EPS)
    return s, b - m * s


def _edge_conv(E, fts, Wtop, Wbot, b1, W2, b2, W3, b3, Wsc, bsc):
    rows = _BB * _N
    inf = jnp.float32(np.inf)
    diag = (jax.lax.broadcasted_iota(jnp.int32, (rows, _N), 0) % _N
            == jax.lax.broadcasted_iota(jnp.int32, (rows, _N), 1))
    E = jnp.where(diag, inf, E)

    f32 = jnp.float32
    u = jnp.dot(fts, Wbot)
    base = jnp.dot(fts, Wtop) - u + b1
    ub = u.astype(jnp.bfloat16)

    acc = jnp.zeros((rows, W3.shape[1]), jnp.float32)
    for k in range(_K):
        mval = jnp.min(E, axis=1, keepdims=True)
        ohb = E == mval
        E = jnp.where(ohb, inf, E)
        oh = ohb.astype(jnp.bfloat16)
        g = jnp.concatenate(
            [jnp.dot(oh[s * _N:(s + 1) * _N], ub[s * _N:(s + 1) * _N],
                     preferred_element_type=f32)
             for s in range(_BB)], axis=0)
        h = jnp.maximum(base + g, 0.0)
        h = jnp.maximum(jnp.dot(h, W2) + b2, 0.0)
        h = jnp.maximum(jnp.dot(h, W3) + b3, 0.0)
        acc = acc + h
    fts_out = acc * jnp.float32(1.0 / _K)
    sc = jnp.dot(fts, Wsc) + bsc
    return jnp.maximum(sc + fts_out, 0.0)


def _body(f_ref, mavg_ref, s0_ref, t0_ref,
          wt0_ref, wb0_ref, b10_ref, w20_ref, b20_ref, w30_ref, b30_ref,
          wsc0_ref, bsc0_ref,
          wt1_ref, wb1_ref, b11_ref, w21_ref, b21_ref, w31_ref, b31_ref,
          wsc1_ref, bsc1_ref,
          fcw_ref, fcb_ref, ow_ref, ob_ref, out_ref):
    rows = _BB * _N
    feats = jnp.reshape(f_ref[...], (rows, _F))
    reduced = jnp.sum(feats, axis=1, keepdims=True)
    mask = (reduced != 0.0).astype(jnp.float32)
    shift = 1e9 * (1.0 - mask)

    fts = feats * s0_ref[...] + t0_ref[...]

    e_list = []
    for s in range(_BB):
        fT = f_ref[s].T
        rT = jnp.sum(fT, axis=0, keepdims=True)
        srow = 1e9 * (rT == 0.0).astype(jnp.float32)
        px = fT[0:1] * jnp.cos(fT[1:2]) + srow
        py = fT[0:1] * jnp.sin(fT[1:2]) + srow
        ptsT = jnp.concatenate([px, py], axis=0)
        mm = jax.lax.dot_general(ptsT, ptsT, (((0,), (0,)), ((), ())))
        e_list.append(px * px + py * py - 2.0 * mm)
    E0 = jnp.concatenate(e_list, axis=0)

    fts = _edge_conv(E0, fts,
                     wt0_ref[...], wb0_ref[...], b10_ref[...],
                     w20_ref[...], b20_ref[...], w30_ref[...], b30_ref[...],
                     wsc0_ref[...], bsc0_ref[...])

    ones_c = jnp.ones((1, fts.shape[1]), jnp.float32)
    e_list = []
    for s in range(_BB):
        P = fts[s * _N:(s + 1) * _N] + shift[s * _N:(s + 1) * _N]
        mm = jax.lax.dot_general(P, P, (((1,), (1,)), ((), ())))
        ra = jax.lax.dot_general(ones_c, P * P, (((1,), (1,)), ((), ())))
        e_list.append(ra - 2.0 * mm)
    E1 = jnp.concatenate(e_list, axis=0)

    fts = _edge_conv(E1, fts,
                     wt1_ref[...], wb1_ref[...], b11_ref[...],
                     w21_ref[...], b21_ref[...], w31_ref[...], b31_ref[...],
                     wsc1_ref[...], bsc1_ref[...])

    fts = fts * mask
    pool = jnp.dot(mavg_ref[...], fts)
    x = jnp.maximum(jnp.dot(pool, fcw_ref[...]) + fcb_ref[...], 0.0)
    logits = jnp.dot(x, ow_ref[...]) + ob_ref[...]
    z = logits - jnp.max(logits, axis=1, keepdims=True)
    e = jnp.exp(z)
    out_ref[...] = e / jnp.sum(e, axis=1, keepdims=True)


def _prep_weights(params):
    s0, t0 = _fold_bn(params["bn0"])
    ws = [s0.reshape(1, -1), t0.reshape(1, -1)]
    for layer in params["layers"]:
        w1, w2, w3 = layer["ws"]
        c_in = w1.shape[0] // 2
        s1, t1 = _fold_bn(layer["bns"][0])
        s2, t2 = _fold_bn(layer["bns"][1])
        s3, t3 = _fold_bn(layer["bns"][2])
        ssc, tsc = _fold_bn(layer["bnsc"])
        bf = jnp.bfloat16
        ws += [w1[:c_in] * s1, w1[c_in:] * s1, t1.reshape(1, -1),
               w2 * s2, t2.reshape(1, -1),
               w3 * s3, t3.reshape(1, -1),
               layer["wsc"] * ssc, tsc.reshape(1, -1)]
    ws += [params["fc_w"], params["fc_b"].reshape(1, -1),
           params["out_w"], params["out_b"].reshape(1, -1)]
    return ws


def kernel(features, params):
    ws = _prep_weights(params)
    mavg = jnp.asarray(np.kron(np.eye(_BB, dtype=np.float32),
                               np.full((1, _N), 1.0 / _N, np.float32)))
    full = lambda a: pl.BlockSpec(a.shape, lambda i: (0,) * a.ndim)
    out = pl.pallas_call(
        _body,
        grid=(_B // _BB,),
        in_specs=[pl.BlockSpec((_BB, _N, _F), lambda i: (i, 0, 0)),
                  full(mavg)] + [full(a) for a in ws],
        out_specs=pl.BlockSpec((_BB, 5), lambda i: (i, 0)),
        out_shape=jax.ShapeDtypeStruct((_B, 5), jnp.float32),
    )(features, mavg, *ws)
    return out


# speedup vs baseline: 1.3742x; 1.3691x over previous
import jax
import jax.numpy as jnp
import numpy as np
from jax.experimental import pallas as pl

_B, _N, _F = 1024, 128, 16
_K = 7
_BB = 16
_EPS = 1e-3


def _fold_bn(p):
    g, b, m, v = p
    s = g * jax.lax.rsqrt(v + _EPS)
    return s, b - m * s


def _edge_conv(E, fts, Wtop, Wbot, b1, W2, b2, W3, b3, Wsc, bsc):
    rows = _BB * _N
    inf = jnp.float32(np.inf)
    diag = (jax.lax.broadcasted_iota(jnp.int32, (rows, _N), 0) % _N
            == jax.lax.broadcasted_iota(jnp.int32, (rows, _N), 1))
    E = jnp.where(diag, inf, E)

    u = jnp.dot(fts, Wbot)
    base = jnp.dot(fts, Wtop) - u + b1

    acc = jnp.zeros((rows, W3.shape[1]), jnp.float32)
    for k in range(_K):
        mval = jnp.min(E, axis=1, keepdims=True)
        ohb = E == mval
        E = jnp.where(ohb, inf, E)
        oh = ohb.astype(jnp.float32)
        g = jnp.concatenate(
            [jnp.dot(oh[s * _N:(s + 1) * _N], u[s * _N:(s + 1) * _N])
             for s in range(_BB)], axis=0)
        h = jnp.maximum(base + g, 0.0)
        h = jnp.maximum(jnp.dot(h, W2) + b2, 0.0)
        h = jnp.maximum(jnp.dot(h, W3) + b3, 0.0)
        acc = acc + h
    fts_out = acc * jnp.float32(1.0 / _K)
    sc = jnp.dot(fts, Wsc) + bsc
    return jnp.maximum(sc + fts_out, 0.0)


def _body(f_ref, mavg_ref, s0_ref, t0_ref,
          wt0_ref, wb0_ref, b10_ref, w20_ref, b20_ref, w30_ref, b30_ref,
          wsc0_ref, bsc0_ref,
          wt1_ref, wb1_ref, b11_ref, w21_ref, b21_ref, w31_ref, b31_ref,
          wsc1_ref, bsc1_ref,
          fcw_ref, fcb_ref, ow_ref, ob_ref, out_ref):
    rows = _BB * _N
    feats = jnp.reshape(f_ref[...], (rows, _F))
    reduced = jnp.sum(feats, axis=1, keepdims=True)
    mask = (reduced != 0.0).astype(jnp.float32)
    shift = 1e9 * (1.0 - mask)

    fts = feats * s0_ref[...] + t0_ref[...]

    e_list = []
    for s in range(_BB):
        fT = f_ref[s].T
        rT = jnp.sum(fT, axis=0, keepdims=True)
        srow = 1e9 * (rT == 0.0).astype(jnp.float32)
        px = fT[0:1] * jnp.cos(fT[1:2]) + srow
        py = fT[0:1] * jnp.sin(fT[1:2]) + srow
        ptsT = jnp.concatenate([px, py], axis=0)
        mm = jax.lax.dot_general(ptsT, ptsT, (((0,), (0,)), ((), ())))
        e_list.append(px * px + py * py - 2.0 * mm)
    E0 = jnp.concatenate(e_list, axis=0)

    fts = _edge_conv(E0, fts,
                     wt0_ref[...], wb0_ref[...], b10_ref[...],
                     w20_ref[...], b20_ref[...], w30_ref[...], b30_ref[...],
                     wsc0_ref[...], bsc0_ref[...])

    ones_c = jnp.ones((1, fts.shape[1]), jnp.float32)
    e_list = []
    for s in range(_BB):
        P = fts[s * _N:(s + 1) * _N] + shift[s * _N:(s + 1) * _N]
        mm = jax.lax.dot_general(P, P, (((1,), (1,)), ((), ())))
        ra = jax.lax.dot_general(ones_c, P * P, (((1,), (1,)), ((), ())))
        e_list.append(ra - 2.0 * mm)
    E1 = jnp.concatenate(e_list, axis=0)

    fts = _edge_conv(E1, fts,
                     wt1_ref[...], wb1_ref[...], b11_ref[...],
                     w21_ref[...], b21_ref[...], w31_ref[...], b31_ref[...],
                     wsc1_ref[...], bsc1_ref[...])

    fts = fts * mask
    pool = jnp.dot(mavg_ref[...], fts)
    x = jnp.maximum(jnp.dot(pool, fcw_ref[...]) + fcb_ref[...], 0.0)
    logits = jnp.dot(x, ow_ref[...]) + ob_ref[...]
    z = logits - jnp.max(logits, axis=1, keepdims=True)
    e = jnp.exp(z)
    out_ref[...] = e / jnp.sum(e, axis=1, keepdims=True)


def _prep_weights(params):
    s0, t0 = _fold_bn(params["bn0"])
    ws = [s0.reshape(1, -1), t0.reshape(1, -1)]
    for layer in params["layers"]:
        w1, w2, w3 = layer["ws"]
        c_in = w1.shape[0] // 2
        s1, t1 = _fold_bn(layer["bns"][0])
        s2, t2 = _fold_bn(layer["bns"][1])
        s3, t3 = _fold_bn(layer["bns"][2])
        ssc, tsc = _fold_bn(layer["bnsc"])
        bf = jnp.bfloat16
        ws += [w1[:c_in] * s1, w1[c_in:] * s1, t1.reshape(1, -1),
               w2 * s2, t2.reshape(1, -1),
               w3 * s3, t3.reshape(1, -1),
               layer["wsc"] * ssc, tsc.reshape(1, -1)]
    ws += [params["fc_w"], params["fc_b"].reshape(1, -1),
           params["out_w"], params["out_b"].reshape(1, -1)]
    return ws


def kernel(features, params):
    ws = _prep_weights(params)
    mavg = jnp.asarray(np.kron(np.eye(_BB, dtype=np.float32),
                               np.full((1, _N), 1.0 / _N, np.float32)))
    full = lambda a: pl.BlockSpec(a.shape, lambda i: (0,) * a.ndim)
    out = pl.pallas_call(
        _body,
        grid=(_B // _BB,),
        in_specs=[pl.BlockSpec((_BB, _N, _F), lambda i: (i, 0, 0)),
                  full(mavg)] + [full(a) for a in ws],
        out_specs=pl.BlockSpec((_BB, 5), lambda i: (i, 0)),
        out_shape=jax.ShapeDtypeStruct((_B, 5), jnp.float32),
    )(features, mavg, *ws)
    return out


# speedup vs baseline: 1.4786x; 1.0760x over previous
import jax
import jax.numpy as jnp
import numpy as np
from jax.experimental import pallas as pl

_B, _N, _F = 1024, 128, 16
_K = 7
_BB = 32
_EPS = 1e-3


def _fold_bn(p):
    g, b, m, v = p
    s = g * jax.lax.rsqrt(v + _EPS)
    return s, b - m * s


def _edge_conv(E, fts, Wtop, Wbot, b1, W2, b2, W3, b3, Wsc, bsc):
    rows = _BB * _N
    inf = jnp.float32(np.inf)
    diag = (jax.lax.broadcasted_iota(jnp.int32, (rows, _N), 0) % _N
            == jax.lax.broadcasted_iota(jnp.int32, (rows, _N), 1))
    E = jnp.where(diag, inf, E)

    u = jnp.dot(fts, Wbot)
    base = jnp.dot(fts, Wtop) - u + b1

    acc = jnp.zeros((rows, W3.shape[1]), jnp.float32)
    for k in range(_K):
        mval = jnp.min(E, axis=1, keepdims=True)
        ohb = E == mval
        E = jnp.where(ohb, inf, E)
        oh = ohb.astype(jnp.float32)
        g = jnp.concatenate(
            [jnp.dot(oh[s * _N:(s + 1) * _N], u[s * _N:(s + 1) * _N])
             for s in range(_BB)], axis=0)
        h = jnp.maximum(base + g, 0.0)
        h = jnp.maximum(jnp.dot(h, W2) + b2, 0.0)
        h = jnp.maximum(jnp.dot(h, W3) + b3, 0.0)
        acc = acc + h
    fts_out = acc * jnp.float32(1.0 / _K)
    sc = jnp.dot(fts, Wsc) + bsc
    return jnp.maximum(sc + fts_out, 0.0)


def _body(f_ref, mavg_ref, s0_ref, t0_ref,
          wt0_ref, wb0_ref, b10_ref, w20_ref, b20_ref, w30_ref, b30_ref,
          wsc0_ref, bsc0_ref,
          wt1_ref, wb1_ref, b11_ref, w21_ref, b21_ref, w31_ref, b31_ref,
          wsc1_ref, bsc1_ref,
          fcw_ref, fcb_ref, ow_ref, ob_ref, out_ref):
    rows = _BB * _N
    feats = jnp.reshape(f_ref[...], (rows, _F))
    reduced = jnp.sum(feats, axis=1, keepdims=True)
    mask = (reduced != 0.0).astype(jnp.float32)
    shift = 1e9 * (1.0 - mask)

    fts = feats * s0_ref[...] + t0_ref[...]

    e_list = []
    for s in range(_BB):
        fT = f_ref[s].T
        rT = jnp.sum(fT, axis=0, keepdims=True)
        srow = 1e9 * (rT == 0.0).astype(jnp.float32)
        px = fT[0:1] * jnp.cos(fT[1:2]) + srow
        py = fT[0:1] * jnp.sin(fT[1:2]) + srow
        ptsT = jnp.concatenate([px, py], axis=0)
        mm = jax.lax.dot_general(ptsT, ptsT, (((0,), (0,)), ((), ())))
        e_list.append(px * px + py * py - 2.0 * mm)
    E0 = jnp.concatenate(e_list, axis=0)

    fts = _edge_conv(E0, fts,
                     wt0_ref[...], wb0_ref[...], b10_ref[...],
                     w20_ref[...], b20_ref[...], w30_ref[...], b30_ref[...],
                     wsc0_ref[...], bsc0_ref[...])

    ones_c = jnp.ones((1, fts.shape[1]), jnp.float32)
    e_list = []
    for s in range(_BB):
        P = fts[s * _N:(s + 1) * _N] + shift[s * _N:(s + 1) * _N]
        mm = jax.lax.dot_general(P, P, (((1,), (1,)), ((), ())))
        ra = jax.lax.dot_general(ones_c, P * P, (((1,), (1,)), ((), ())))
        e_list.append(ra - 2.0 * mm)
    E1 = jnp.concatenate(e_list, axis=0)

    fts = _edge_conv(E1, fts,
                     wt1_ref[...], wb1_ref[...], b11_ref[...],
                     w21_ref[...], b21_ref[...], w31_ref[...], b31_ref[...],
                     wsc1_ref[...], bsc1_ref[...])

    fts = fts * mask
    pool = jnp.dot(mavg_ref[...], fts)
    x = jnp.maximum(jnp.dot(pool, fcw_ref[...]) + fcb_ref[...], 0.0)
    logits = jnp.dot(x, ow_ref[...]) + ob_ref[...]
    z = logits - jnp.max(logits, axis=1, keepdims=True)
    e = jnp.exp(z)
    out_ref[...] = e / jnp.sum(e, axis=1, keepdims=True)


def _prep_weights(params):
    s0, t0 = _fold_bn(params["bn0"])
    ws = [s0.reshape(1, -1), t0.reshape(1, -1)]
    for layer in params["layers"]:
        w1, w2, w3 = layer["ws"]
        c_in = w1.shape[0] // 2
        s1, t1 = _fold_bn(layer["bns"][0])
        s2, t2 = _fold_bn(layer["bns"][1])
        s3, t3 = _fold_bn(layer["bns"][2])
        ssc, tsc = _fold_bn(layer["bnsc"])
        bf = jnp.bfloat16
        ws += [w1[:c_in] * s1, w1[c_in:] * s1, t1.reshape(1, -1),
               w2 * s2, t2.reshape(1, -1),
               w3 * s3, t3.reshape(1, -1),
               layer["wsc"] * ssc, tsc.reshape(1, -1)]
    ws += [params["fc_w"], params["fc_b"].reshape(1, -1),
           params["out_w"], params["out_b"].reshape(1, -1)]
    return ws


def kernel(features, params):
    ws = _prep_weights(params)
    mavg = jnp.asarray(np.kron(np.eye(_BB, dtype=np.float32),
                               np.full((1, _N), 1.0 / _N, np.float32)))
    full = lambda a: pl.BlockSpec(a.shape, lambda i: (0,) * a.ndim)
    out = pl.pallas_call(
        _body,
        grid=(_B // _BB,),
        in_specs=[pl.BlockSpec((_BB, _N, _F), lambda i: (i, 0, 0)),
                  full(mavg)] + [full(a) for a in ws],
        out_specs=pl.BlockSpec((_BB, 5), lambda i: (i, 0)),
        out_shape=jax.ShapeDtypeStruct((_B, 5), jnp.float32),
    )(features, mavg, *ws)
    return out
